# Initial kernel scaffold; baseline (speedup 1.0000x reference)
import jax
import jax.numpy as jnp
from jax.experimental import pallas as pl


def kernel(indices, weight):
    raise NotImplementedError("write your pallas kernel here")


# speedup vs baseline: 1.2335x; 1.2335x over previous
import functools

import jax
import jax.numpy as jnp
from jax import lax
from jax.experimental import pallas as pl
from jax.experimental.pallas import tpu as pltpu
from jax.experimental.pallas import tpu_sc as plsc

_D = 512
_B = 4096 * 20

_info = plsc.get_sparse_core_info()
_NC, _NS = _info.num_cores, _info.num_subcores
_NW = _NC * _NS
_BPW = _B // _NW
_CH = 64
_NCHUNK = _BPW // _CH


def _make_gather():
    mesh = plsc.VectorSubcoreMesh(core_axis_name="c", subcore_axis_name="s")

    @functools.partial(
        pl.kernel,
        mesh=mesh,
        out_type=jax.ShapeDtypeStruct((_B, _D), jnp.float32),
        scratch_types=[
            pltpu.VMEM((_NCHUNK, _CH), jnp.int32),
            pltpu.VMEM((_CH, _D), jnp.float32),
            pltpu.SemaphoreType.DMA,
        ],
    )
    def gather_k(idx_hbm, table_hbm, out_hbm, idx_v, buf, sem):
        wid = lax.axis_index("s") * _NC + lax.axis_index("c")
        base = wid * _BPW
        pltpu.sync_copy(idx_hbm.at[pl.ds(wid * _NCHUNK, _NCHUNK)], idx_v)

        def body(c, carry):
            pltpu.async_copy(table_hbm.at[idx_v.at[c]], buf, sem).wait()
            pltpu.sync_copy(buf, out_hbm.at[pl.ds(base + c * _CH, _CH)])
            return carry

        lax.fori_loop(0, _NCHUNK, body, 0)

    return gather_k


_gather = _make_gather()


@jax.jit
def kernel(indices, weight):
    idx = indices.reshape(_NW * _NCHUNK, _CH).astype(jnp.int32)
    out = _gather(idx, weight)
    return out.reshape(indices.shape[0], indices.shape[1], _D)


# speedup vs baseline: 1.3085x; 1.0609x over previous
import functools

import jax
import jax.numpy as jnp
from jax import lax
from jax.experimental import pallas as pl
from jax.experimental.pallas import tpu as pltpu
from jax.experimental.pallas import tpu_sc as plsc

_D = 512
_B = 4096 * 20

_info = plsc.get_sparse_core_info()
_NC, _NS = _info.num_cores, _info.num_subcores
_NW = _NC * _NS
_BPW = _B // _NW
_CH = 80
_NCHUNK = _BPW // _CH
_NPAIR = _NCHUNK // 2


def _make_gather():
    mesh = plsc.VectorSubcoreMesh(core_axis_name="c", subcore_axis_name="s")

    @functools.partial(
        pl.kernel,
        mesh=mesh,
        out_type=jax.ShapeDtypeStruct((_B, _D), jnp.float32),
        scratch_types=[
            pltpu.VMEM((_NCHUNK, _CH), jnp.int32),
            pltpu.VMEM((_CH, _D), jnp.float32),
            pltpu.VMEM((_CH, _D), jnp.float32),
            pltpu.SemaphoreType.DMA,
            pltpu.SemaphoreType.DMA,
        ],
    )
    def gather_k(idx_hbm, table_hbm, out_hbm, idx_v, buf0, buf1, sem0, sem1):
        wid = lax.axis_index("s") * _NC + lax.axis_index("c")
        base = wid * _BPW
        pltpu.sync_copy(idx_hbm.at[pl.ds(wid * _NCHUNK, _NCHUNK)], idx_v)

        pltpu.async_copy(table_hbm.at[idx_v.at[0]], buf0, sem0)

        def body(p, carry):
            c0 = 2 * p
            pltpu.async_copy(table_hbm.at[idx_v.at[c0 + 1]], buf1, sem1)
            pltpu.make_async_copy(table_hbm.at[idx_v.at[c0]], buf0, sem0).wait()
            pltpu.sync_copy(buf0, out_hbm.at[pl.ds(base + c0 * _CH, _CH)])

            @pl.when(p + 1 < _NPAIR)
            def _():
                pltpu.async_copy(table_hbm.at[idx_v.at[c0 + 2]], buf0, sem0)

            pltpu.make_async_copy(table_hbm.at[idx_v.at[c0 + 1]], buf1, sem1).wait()
            pltpu.sync_copy(buf1, out_hbm.at[pl.ds(base + (c0 + 1) * _CH, _CH)])
            return carry

        lax.fori_loop(0, _NPAIR, body, 0)

    return gather_k


_gather = _make_gather()


@jax.jit
def kernel(indices, weight):
    idx = indices.reshape(_NW * _NCHUNK, _CH).astype(jnp.int32)
    out = _gather(idx, weight)
    return out.reshape(indices.shape[0], indices.shape[1], _D)


# speedup vs baseline: 1.3114x; 1.0022x over previous
import functools

import jax
import jax.numpy as jnp
from jax import lax
from jax.experimental import pallas as pl
from jax.experimental.pallas import tpu as pltpu
from jax.experimental.pallas import tpu_sc as plsc

_D = 512
_B = 4096 * 20

_info = plsc.get_sparse_core_info()
_NC, _NS = _info.num_cores, _info.num_subcores
_NW = _NC * _NS
_BPW = _B // _NW
_CH = 40
_NCHUNK = _BPW // _CH
_NB = 4
_LOOK = 2
_NROUND = _NCHUNK // _NB


def _make_gather():
    mesh = plsc.VectorSubcoreMesh(core_axis_name="c", subcore_axis_name="s")

    @functools.partial(
        pl.kernel,
        mesh=mesh,
        out_type=jax.ShapeDtypeStruct((_B, _D), jnp.float32),
        scratch_types=[
            pltpu.VMEM((_NCHUNK, _CH), jnp.int32),
            pltpu.VMEM((_CH, _D), jnp.float32),
            pltpu.VMEM((_CH, _D), jnp.float32),
            pltpu.VMEM((_CH, _D), jnp.float32),
            pltpu.VMEM((_CH, _D), jnp.float32),
            pltpu.SemaphoreType.DMA,
            pltpu.SemaphoreType.DMA,
            pltpu.SemaphoreType.DMA,
            pltpu.SemaphoreType.DMA,
            pltpu.SemaphoreType.DMA,
            pltpu.SemaphoreType.DMA,
            pltpu.SemaphoreType.DMA,
            pltpu.SemaphoreType.DMA,
        ],
    )
    def gather_k(idx_hbm, table_hbm, out_hbm, idx_v,
                 b0, b1, b2, b3, g0, g1, g2, g3, w0, w1, w2, w3):
        buf = [b0, b1, b2, b3]
        gsem = [g0, g1, g2, g3]
        wsem = [w0, w1, w2, w3]

        wid = lax.axis_index("s") * _NC + lax.axis_index("c")
        base = wid * _BPW
        pltpu.sync_copy(idx_hbm.at[pl.ds(wid * _NCHUNK, _NCHUNK)], idx_v)

        def start_gather(c, b):
            pltpu.async_copy(table_hbm.at[idx_v.at[c]], buf[b], gsem[b])

        def wait_gather(c, b):
            pltpu.make_async_copy(table_hbm.at[idx_v.at[c]], buf[b],
                                  gsem[b]).wait()

        def start_wb(c, b):
            pltpu.async_copy(buf[b], out_hbm.at[pl.ds(base + c * _CH, _CH)],
                             wsem[b])

        def wait_wb(c, b):
            pltpu.make_async_copy(buf[b], out_hbm.at[pl.ds(base + c * _CH, _CH)],
                                  wsem[b]).wait()

        start_gather(0, 0)
        start_gather(1, 1)

        for b in range(_NB):
            wait_gather(b, b)
            start_wb(b, b)
            cn = b + _LOOK
            bn = cn % _NB
            if b >= _LOOK:
                wait_wb(cn - _NB, bn)
            start_gather(cn, bn)

        def round_body(p, carry):
            for b in range(_NB):
                c = _NB * p + b
                cn = c + _LOOK
                bn = (b + _LOOK) % _NB
                wait_gather(c, b)
                start_wb(c, b)
                wait_wb(cn - _NB, bn)
                start_gather(cn, bn)
            return carry

        lax.fori_loop(1, _NROUND - 1, round_body, 0)

        for b in range(_NB):
            c = _NB * (_NROUND - 1) + b
            cn = c + _LOOK
            bn = (b + _LOOK) % _NB
            wait_gather(c, b)
            start_wb(c, b)
            if cn < _NCHUNK:
                wait_wb(cn - _NB, bn)
                start_gather(cn, bn)

        for b in range(_NB):
            wait_wb(_NCHUNK - _NB + b, b)

    return gather_k


_gather = _make_gather()


@jax.jit
def kernel(indices, weight):
    idx = indices.reshape(_NW * _NCHUNK, _CH).astype(jnp.int32)
    out = _gather(idx, weight)
    return out.reshape(indices.shape[0], indices.shape[1], _D)


# speedup vs baseline: 2.0109x; 1.5334x over previous
import functools

import jax
import jax.numpy as jnp
from jax import lax
from jax.experimental import pallas as pl
from jax.experimental.pallas import tpu as pltpu
from jax.experimental.pallas import tpu_sc as plsc

_D = 512
_G = 4096
_GW = 20
_GP = 24

_info = plsc.get_sparse_core_info()
_NC, _NS = _info.num_cores, _info.num_subcores
_NW = _NC * _NS
_GPW = _G // _NW
_CPG = 2
_RPC = _CPG * _GP
_NCHUNK = _GPW // _CPG
_NB = 4
_LOOK = 2
_NROUND = _NCHUNK // _NB


def _make_gather():
    mesh = plsc.VectorSubcoreMesh(core_axis_name="c", subcore_axis_name="s")

    @functools.partial(
        pl.kernel,
        mesh=mesh,
        out_type=jax.ShapeDtypeStruct((_G * _GP, _D), jnp.float32),
        scratch_types=[
            pltpu.VMEM((_NCHUNK, _RPC), jnp.int32),
            pltpu.VMEM((_RPC, _D), jnp.float32),
            pltpu.VMEM((_RPC, _D), jnp.float32),
            pltpu.VMEM((_RPC, _D), jnp.float32),
            pltpu.VMEM((_RPC, _D), jnp.float32),
            pltpu.SemaphoreType.DMA,
            pltpu.SemaphoreType.DMA,
            pltpu.SemaphoreType.DMA,
            pltpu.SemaphoreType.DMA,
            pltpu.SemaphoreType.DMA,
            pltpu.SemaphoreType.DMA,
            pltpu.SemaphoreType.DMA,
            pltpu.SemaphoreType.DMA,
        ],
    )
    def gather_k(idx_hbm, table_hbm, out_hbm, idx_v,
                 b0, b1, b2, b3, g0, g1, g2, g3, w0, w1, w2, w3):
        buf = [b0, b1, b2, b3]
        gsem = [g0, g1, g2, g3]
        wsem = [w0, w1, w2, w3]

        wid = lax.axis_index("s") * _NC + lax.axis_index("c")
        rbase = wid * _NCHUNK * _RPC
        pltpu.sync_copy(idx_hbm.at[pl.ds(wid * _NCHUNK, _NCHUNK)], idx_v)

        def start_gather(c, b):
            pltpu.async_copy(table_hbm.at[idx_v.at[c]], buf[b], gsem[b])

        def wait_gather(c, b):
            pltpu.make_async_copy(table_hbm.at[idx_v.at[c]], buf[b],
                                  gsem[b]).wait()

        def start_wb(c, b):
            pltpu.async_copy(buf[b], out_hbm.at[pl.ds(rbase + c * _RPC, _RPC)],
                             wsem[b])

        def wait_wb(c, b):
            pltpu.make_async_copy(buf[b],
                                  out_hbm.at[pl.ds(rbase + c * _RPC, _RPC)],
                                  wsem[b]).wait()

        start_gather(0, 0)
        start_gather(1, 1)

        for b in range(_NB):
            wait_gather(b, b)
            start_wb(b, b)
            cn = b + _LOOK
            bn = cn % _NB
            if b >= _LOOK:
                wait_wb(cn - _NB, bn)
            start_gather(cn, bn)

        def round_body(p, carry):
            for b in range(_NB):
                c = _NB * p + b
                cn = c + _LOOK
                bn = (b + _LOOK) % _NB
                wait_gather(c, b)
                start_wb(c, b)
                wait_wb(cn - _NB, bn)
                start_gather(cn, bn)
            return carry

        lax.fori_loop(1, _NROUND - 1, round_body, 0)

        for b in range(_NB):
            c = _NB * (_NROUND - 1) + b
            cn = c + _LOOK
            bn = (b + _LOOK) % _NB
            wait_gather(c, b)
            start_wb(c, b)
            if cn < _NCHUNK:
                wait_wb(cn - _NB, bn)
                start_gather(cn, bn)

        for b in range(_NB):
            wait_wb(_NCHUNK - _NB + b, b)

    return gather_k


_gather = _make_gather()


@jax.jit
def kernel(indices, weight):
    idx = indices.astype(jnp.int32)
    idx24 = jnp.concatenate([idx, idx[:, :_GP - _GW]], axis=1)
    idx_chunks = idx24.reshape(_NW * _NCHUNK, _RPC)
    out = _gather(idx_chunks, weight)
    return out.reshape(_G, _GP, _D)[:, :_GW, :]
